# Initial kernel scaffold; baseline (speedup 1.0000x reference)
import jax
import jax.numpy as jnp
from jax.experimental import pallas as pl


def kernel(x, Wg, Wu, Wd, Wr):
    raise NotImplementedError("write your pallas kernel here")


# speedup vs baseline: 1.1776x; 1.1776x over previous
import jax
import jax.numpy as jnp
from jax.experimental import pallas as pl
from jax.experimental.pallas import tpu as pltpu

T, D, F, E = 2048, 1024, 2048, 8
FT = 512
NF = F // FT
TAU = 1.0


def _moe_body(x_ref, wr_ref, wg_ref, wu_ref, wd_ref, out_ref, gate_ref):
    e = pl.program_id(0)
    f = pl.program_id(1)
    first = (e == 0) & (f == 0)

    @pl.when(first)
    def _():
        logits = jnp.dot(x_ref[:], wr_ref[:], preferred_element_type=jnp.float32)
        logits = logits / TAU
        m = jnp.max(logits, axis=1, keepdims=True)
        p = jnp.exp(logits - m)
        gate_ref[:] = p / jnp.sum(p, axis=1, keepdims=True)

    xb = x_ref[:]
    g = jnp.dot(xb, wg_ref[0], preferred_element_type=jnp.float32)
    u = jnp.dot(xb, wu_ref[0], preferred_element_type=jnp.float32)
    act = (g * jax.nn.sigmoid(g)) * u
    onehot = (jax.lax.broadcasted_iota(jnp.int32, (E, 1), 0) == e).astype(
        jnp.float32
    )
    gcol = jnp.dot(gate_ref[:], onehot, preferred_element_type=jnp.float32)
    act = act * gcol
    part = jnp.dot(act, wd_ref[0], preferred_element_type=jnp.float32)

    @pl.when(first)
    def _():
        out_ref[:] = part

    @pl.when(~first)
    def _():
        out_ref[:] = out_ref[:] + part


def kernel(x, Wg, Wu, Wd, Wr):
    return pl.pallas_call(
        _moe_body,
        grid=(E, NF),
        in_specs=[
            pl.BlockSpec((T, D), lambda e, f: (0, 0)),
            pl.BlockSpec((D, E), lambda e, f: (0, 0)),
            pl.BlockSpec((1, D, FT), lambda e, f: (e, 0, f)),
            pl.BlockSpec((1, D, FT), lambda e, f: (e, 0, f)),
            pl.BlockSpec((1, FT, D), lambda e, f: (e, f, 0)),
        ],
        out_specs=pl.BlockSpec((T, D), lambda e, f: (0, 0)),
        out_shape=jax.ShapeDtypeStruct((T, D), jnp.float32),
        scratch_shapes=[pltpu.VMEM((T, E), jnp.float32)],
    )(x, Wr, Wg, Wu, Wd)
